# baseline (speedup 1.0000x reference)
import functools

import jax
import jax.numpy as jnp
from jax import lax
from jax.experimental import pallas as pl
from jax.experimental.pallas import tpu as pltpu
from jax.experimental.pallas import tpu_sc as plsc

NC, NS = 2, 16
NW = NC * NS
V = 1000001
D = 32
B = 16384 * 50
VT_FULL = V // 128
V_TAIL = V - VT_FULL * 128
VPAD = 1000004
SLAB = 256
SPW = 7808 // (SLAB // 128) // NW
REM1 = VT_FULL - 7808
RT = 16384 // 128
B_PER_W = B // NW
CH = 640
NCH = B_PER_W // CH
TPW2 = B_PER_W // 128
TILE = 128 * D

_mesh = plsc.VectorSubcoreMesh(core_axis_name="c", subcore_axis_name="s")


def _wid():
    return lax.axis_index("s") * NC + lax.axis_index("c")


def _iota():
    return lax.iota(jnp.int32, 16)


@functools.partial(
    pl.kernel,
    out_type=(
        jax.ShapeDtypeStruct((VPAD * D,), jnp.float32),
        jax.ShapeDtypeStruct((B,), jnp.int32),
    ),
    mesh=_mesh,
    compiler_params=pltpu.CompilerParams(
        use_tc_tiling_on_sc=True, needs_layout_passes=False
    ),
    scratch_types=[
        pltpu.VMEM((D, SLAB), jnp.float32),
        pltpu.VMEM((D, SLAB), jnp.float32),
        pltpu.VMEM((SLAB * D,), jnp.float32),
        pltpu.VMEM((SLAB * D,), jnp.float32),
        pltpu.VMEM((50, 128), jnp.int32),
        pltpu.VMEM((128 * 50,), jnp.int32),
        pltpu.SemaphoreType.DMA,
        pltpu.SemaphoreType.DMA,
    ],
)
def _k1(tableT, idxT, tail1d, packed, flatidx,
        tb0, tb1, pb0, pb1, ibuf, obuf, isem, osem):
    wid = _wid()
    i16 = _iota()
    perms = [(i16 + c) & 15 for c in range(16)]
    i32v = i16 * D
    widx = [i32v + perms[c] for c in range(16)]
    i50x = i16 * 50
    tbufs = (tb0, tb1)
    pbufs = (pb0, pb1)
    base = wid * SPW

    def fire_in(t, s):
        pltpu.async_copy(tableT.at[:, pl.ds(t * SLAB, SLAB)], tbufs[s], isem)

    def wait_in(s):
        pltpu.make_async_copy(
            tableT.at[:, pl.ds(0, SLAB)], tbufs[s], isem
        ).wait()

    def fire_out(t, s):
        pltpu.async_copy(
            pbufs[s], packed.at[pl.ds(t * (SLAB * D), SLAB * D)], osem
        )

    def wait_out(s):
        pltpu.make_async_copy(
            pbufs[s], packed.at[pl.ds(0, SLAB * D)], osem
        ).wait()

    def transpose_slab(s, nlanes):
        @pl.loop(0, nlanes, step=16)
        def _(v0):
            cols = i16 + v0
            for f0 in (0, 16):
                for c in range(16):
                    x = plsc.load_gather(tbufs[s], [perms[c] + f0, cols])
                    plsc.store_scatter(
                        pbufs[s], [widx[c] + (v0 * D + f0)], x
                    )

    for s in range(2):
        fire_in(base + s, s)
    for s in range(2):
        wait_in(s)
        transpose_slab(s, SLAB)
        fire_out(base + s, s)
        fire_in(base + s + 2, s)

    @pl.loop(2, SPW - 2, step=2)
    def _(j):
        for s in range(2):
            t = base + j + s
            wait_in(s)
            wait_out(s)
            transpose_slab(s, SLAB)
            fire_out(t, s)
            fire_in(t + 2, s)

    for s in range(2):
        wait_in(s)
        wait_out(s)
        transpose_slab(s, SLAB)
        fire_out(base + (SPW - 2) + s, s)
    for s in range(2):
        wait_out(s)

    @pl.when(wid < REM1)
    def _():
        t = NW * SPW * (SLAB // 128) + wid
        pltpu.sync_copy(
            tableT.at[:, pl.ds(t * 128, 128)], tb0.at[:, pl.ds(0, 128)]
        )
        transpose_slab(0, 128)
        pltpu.sync_copy(
            pb0.at[pl.ds(0, TILE)], packed.at[pl.ds(t * TILE, TILE)]
        )

    @pl.when(wid == REM1)
    def _():
        pltpu.sync_copy(tail1d, pb0.at[pl.ds(0, V_TAIL * D)])
        pltpu.sync_copy(
            pb0.at[pl.ds(0, (VPAD - VT_FULL * 128) * D)],
            packed.at[pl.ds(VT_FULL * 128 * D, (VPAD - VT_FULL * 128) * D)],
        )

    @pl.loop(0, RT // NW)
    def _(j):
        t = wid * (RT // NW) + j
        pltpu.sync_copy(idxT.at[:, pl.ds(t * 128, 128)], ibuf)

        @pl.loop(0, 8)
        def _(vv0):
            for c in range(50):
                x = ibuf[c, pl.ds(vv0 * 16, 16)]
                plsc.store_scatter(obuf, [i50x + (vv0 * (16 * 50) + c)], x)

        pltpu.sync_copy(obuf, flatidx.at[pl.ds(t * 6400, 6400)])


@functools.partial(
    pl.kernel,
    out_type=jax.ShapeDtypeStruct((B * D,), jnp.float32),
    mesh=_mesh,
    compiler_params=pltpu.CompilerParams(
        use_tc_tiling_on_sc=False, needs_layout_passes=False
    ),
    scratch_types=[
        pltpu.VMEM((CH,), jnp.int32),
        pltpu.VMEM((CH,), jnp.int32),
        pltpu.VMEM((CH, D), jnp.float32),
        pltpu.VMEM((CH, D), jnp.float32),
        pltpu.VMEM((CH * D,), jnp.float32),
        pltpu.VMEM((CH * D,), jnp.float32),
        pltpu.SemaphoreType.DMA,
        pltpu.SemaphoreType.DMA,
        pltpu.SemaphoreType.DMA,
    ],
)
def _k2(flatidx, table2d, out1d, ix0, ix1, gb0, gb1, tp0, tp1,
        gsem, osem, xsem):
    wid = _wid()
    base = wid * B_PER_W
    i16 = _iota()
    perms = [(i16 + c) & 15 for c in range(16)]
    GSTR = (CH // 128) * 1024
    dsts = [((perms[c] >> 3) * GSTR) + ((perms[c] & 7) << 7) for c in range(16)]
    ixs = (ix0, ix1)
    gbs = (gb0, gb1)
    tps = (tp0, tp1)

    def fire_idx(j, b):
        pltpu.async_copy(flatidx.at[pl.ds(base + j * CH, CH)], ixs[b], xsem)

    def fire_gather(b):
        pltpu.make_async_copy(
            flatidx.at[pl.ds(0, CH)], ixs[b], xsem
        ).wait()
        pltpu.async_copy(table2d.at[ixs[b]], gbs[b], gsem)

    def drain_gather(b):
        pltpu.make_async_copy(table2d.at[ixs[b]], gbs[b], gsem).wait()

    def transpose_chunk(b):
        @pl.loop(0, CH, step=16)
        def _(k0):
            rows = i16 + k0
            koff = i16 + (((k0 >> 7) * 1024) + (k0 & 127))
            koff8 = koff + 2 * GSTR
            for f0 in (0, 16):
                for c in range(16):
                    x = plsc.load_gather(gbs[b], [rows, perms[c] + f0])
                    plsc.store_scatter(
                        tps[b], [dsts[c] + (koff if f0 == 0 else koff8)], x
                    )

    def fire_outs(j, b):
        t0 = wid * TPW2 + j * (CH // 128)
        for g in range(4):
            pltpu.async_copy(
                tps[b].at[pl.ds(g * GSTR, GSTR)],
                out1d.at[pl.ds((g * 6400 + t0) * 1024, GSTR)],
                osem,
            )

    def wait_outs(b):
        for g in range(4):
            pltpu.make_async_copy(
                tps[b].at[pl.ds(0, GSTR)], out1d.at[pl.ds(0, GSTR)], osem
            ).wait()

    for b in range(2):
        fire_idx(b, b)
    for b in range(2):
        fire_gather(b)
    for b in range(2):
        drain_gather(b)
        fire_idx(b + 2, b)
        transpose_chunk(b)
        fire_outs(b, b)
        fire_gather(b)

    @pl.loop(2, NCH - 2, step=2)
    def _(j):
        for b in range(2):
            drain_gather(b)
            fire_idx(j + b + 2, b)
            wait_outs(b)
            transpose_chunk(b)
            fire_outs(j + b, b)
            fire_gather(b)

    for b in range(2):
        drain_gather(b)
        wait_outs(b)
        transpose_chunk(b)
        fire_outs(NCH - 2 + b, b)
    for b in range(2):
        wait_outs(b)


def kernel(input, table):
    idxT = input.astype(jnp.int32).T
    tableT = table.T
    tail1d = table[V - V_TAIL:, :].reshape(-1)
    packed, flatidx = _k1(tableT, idxT, tail1d)
    out1d = _k2(flatidx, packed.reshape(VPAD, D))
    return out1d.reshape(4, 6400, 8, 128).transpose(1, 3, 0, 2).reshape(B, D)


# speedup vs baseline: 1.0653x; 1.0653x over previous
import functools

import jax
import jax.numpy as jnp
from jax import lax
from jax.experimental import pallas as pl
from jax.experimental.pallas import tpu as pltpu
from jax.experimental.pallas import tpu_sc as plsc

NC, NS = 2, 16
NW = NC * NS
V = 1000001
D = 32
B = 16384 * 50
VT_FULL = V // 128
V_TAIL = V - VT_FULL * 128
VPAD = 1000004
SLAB = 256
SPW = 7808 // (SLAB // 128) // NW
REM1 = VT_FULL - 7808
RT = 16384 // 128
B_PER_W = B // NW
CH = 512
NCH = B_PER_W // CH
TPW2 = B_PER_W // 128
TILE = 128 * D

_mesh = plsc.VectorSubcoreMesh(core_axis_name="c", subcore_axis_name="s")


def _wid():
    return lax.axis_index("s") * NC + lax.axis_index("c")


def _iota():
    return lax.iota(jnp.int32, 16)


@functools.partial(
    pl.kernel,
    out_type=(
        jax.ShapeDtypeStruct((VPAD * D,), jnp.float32),
        jax.ShapeDtypeStruct((B,), jnp.int32),
    ),
    mesh=_mesh,
    compiler_params=pltpu.CompilerParams(
        use_tc_tiling_on_sc=True, needs_layout_passes=False
    ),
    scratch_types=[
        pltpu.VMEM((D, SLAB), jnp.float32),
        pltpu.VMEM((D, SLAB), jnp.float32),
        pltpu.VMEM((SLAB * D,), jnp.float32),
        pltpu.VMEM((SLAB * D,), jnp.float32),
        pltpu.VMEM((50, 128), jnp.int32),
        pltpu.VMEM((128 * 50,), jnp.int32),
        pltpu.SemaphoreType.DMA,
        pltpu.SemaphoreType.DMA,
    ],
)
def _k1(tableT, idxT, tail1d, packed, flatidx,
        tb0, tb1, pb0, pb1, ibuf, obuf, isem, osem):
    wid = _wid()
    i16 = _iota()
    perms = [(i16 + c) & 15 for c in range(16)]
    i32v = i16 * D
    widx = [i32v + perms[c] for c in range(16)]
    i50x = i16 * 50
    tbufs = (tb0, tb1)
    pbufs = (pb0, pb1)
    base = wid * SPW

    def fire_in(t, s):
        pltpu.async_copy(tableT.at[:, pl.ds(t * SLAB, SLAB)], tbufs[s], isem)

    def wait_in(s):
        pltpu.make_async_copy(
            tableT.at[:, pl.ds(0, SLAB)], tbufs[s], isem
        ).wait()

    def fire_out(t, s):
        pltpu.async_copy(
            pbufs[s], packed.at[pl.ds(t * (SLAB * D), SLAB * D)], osem
        )

    def wait_out(s):
        pltpu.make_async_copy(
            pbufs[s], packed.at[pl.ds(0, SLAB * D)], osem
        ).wait()

    def transpose_slab(s, nlanes):
        @pl.loop(0, nlanes, step=16)
        def _(v0):
            cols = i16 + v0
            for f0 in (0, 16):
                for c in range(16):
                    x = plsc.load_gather(tbufs[s], [perms[c] + f0, cols])
                    plsc.store_scatter(
                        pbufs[s], [widx[c] + (v0 * D + f0)], x
                    )

    for s in range(2):
        fire_in(base + s, s)
    for s in range(2):
        wait_in(s)
        transpose_slab(s, SLAB)
        fire_out(base + s, s)
        fire_in(base + s + 2, s)

    @pl.loop(2, SPW - 2, step=2)
    def _(j):
        for s in range(2):
            t = base + j + s
            wait_in(s)
            wait_out(s)
            transpose_slab(s, SLAB)
            fire_out(t, s)
            fire_in(t + 2, s)

    for s in range(2):
        wait_in(s)
        wait_out(s)
        transpose_slab(s, SLAB)
        fire_out(base + (SPW - 2) + s, s)
    for s in range(2):
        wait_out(s)

    @pl.when(wid < REM1)
    def _():
        t = NW * SPW * (SLAB // 128) + wid
        pltpu.sync_copy(
            tableT.at[:, pl.ds(t * 128, 128)], tb0.at[:, pl.ds(0, 128)]
        )
        transpose_slab(0, 128)
        pltpu.sync_copy(
            pb0.at[pl.ds(0, TILE)], packed.at[pl.ds(t * TILE, TILE)]
        )

    @pl.when(wid == REM1)
    def _():
        pltpu.sync_copy(tail1d, pb0.at[pl.ds(0, V_TAIL * D)])
        pltpu.sync_copy(
            pb0.at[pl.ds(0, (VPAD - VT_FULL * 128) * D)],
            packed.at[pl.ds(VT_FULL * 128 * D, (VPAD - VT_FULL * 128) * D)],
        )

    @pl.loop(0, RT // NW)
    def _(j):
        t = wid * (RT // NW) + j
        pltpu.sync_copy(idxT.at[:, pl.ds(t * 128, 128)], ibuf)

        @pl.loop(0, 8)
        def _(vv0):
            for c in range(50):
                x = ibuf[c, pl.ds(vv0 * 16, 16)]
                plsc.store_scatter(obuf, [i50x + (vv0 * (16 * 50) + c)], x)

        pltpu.sync_copy(obuf, flatidx.at[pl.ds(t * 6400, 6400)])


@functools.partial(
    pl.kernel,
    out_type=jax.ShapeDtypeStruct((B * D,), jnp.float32),
    mesh=_mesh,
    compiler_params=pltpu.CompilerParams(
        use_tc_tiling_on_sc=False, needs_layout_passes=False
    ),
    scratch_types=[
        pltpu.VMEM((CH,), jnp.int32),
        pltpu.VMEM((CH,), jnp.int32),
        pltpu.VMEM((CH, D), jnp.float32),
        pltpu.VMEM((CH, D), jnp.float32),
        pltpu.VMEM((CH * D,), jnp.float32),
        pltpu.VMEM((CH * D,), jnp.float32),
        pltpu.SemaphoreType.DMA,
        pltpu.SemaphoreType.DMA,
        pltpu.SemaphoreType.DMA,
    ],
)
def _k2(flatidx, table2d, out1d, ix0, ix1, gb0, gb1, tp0, tp1,
        gsem, osem, xsem):
    wid = _wid()
    base = wid * B_PER_W
    i16 = _iota()
    perms = [(i16 + c) & 15 for c in range(16)]
    GSTR = (CH // 128) * 1024
    dsts = [((perms[c] >> 3) * GSTR) + ((perms[c] & 7) << 7) for c in range(16)]
    ixs = (ix0, ix1)
    gbs = (gb0, gb1)
    tps = (tp0, tp1)

    def fire_idx(j, b):
        pltpu.async_copy(flatidx.at[pl.ds(base + j * CH, CH)], ixs[b], xsem)

    def fire_gather(b):
        pltpu.make_async_copy(
            flatidx.at[pl.ds(0, CH)], ixs[b], xsem
        ).wait()
        pltpu.async_copy(table2d.at[ixs[b]], gbs[b], gsem)

    def drain_gather(b):
        pltpu.make_async_copy(table2d.at[ixs[b]], gbs[b], gsem).wait()

    def transpose_chunk(b):
        @pl.loop(0, CH, step=16)
        def _(k0):
            rows = i16 + k0
            koff = i16 + (((k0 >> 7) * 1024) + (k0 & 127))
            koff8 = koff + 2 * GSTR
            for f0 in (0, 16):
                for c in range(16):
                    x = plsc.load_gather(gbs[b], [rows, perms[c] + f0])
                    plsc.store_scatter(
                        tps[b], [dsts[c] + (koff if f0 == 0 else koff8)], x
                    )

    def fire_outs(j, b):
        t0 = wid * TPW2 + j * (CH // 128)
        for g in range(4):
            pltpu.async_copy(
                tps[b].at[pl.ds(g * GSTR, GSTR)],
                out1d.at[pl.ds((g * 6400 + t0) * 1024, GSTR)],
                osem,
            )

    def wait_outs(b):
        for g in range(4):
            pltpu.make_async_copy(
                tps[b].at[pl.ds(0, GSTR)], out1d.at[pl.ds(0, GSTR)], osem
            ).wait()

    for b in range(2):
        fire_idx(b, b)
    for b in range(2):
        fire_gather(b)
    for b in range(2):
        drain_gather(b)
        fire_idx(b + 2, b)
        transpose_chunk(b)
        fire_outs(b, b)
        fire_gather(b)

    @pl.loop(2, NCH - 2, step=2)
    def _(j):
        for b in range(2):
            drain_gather(b)
            fire_idx(j + b + 2, b)
            wait_outs(b)
            transpose_chunk(b)
            fire_outs(j + b, b)
            fire_gather(b)

    for b in range(2):
        drain_gather(b)
        wait_outs(b)
        transpose_chunk(b)
        fire_outs(NCH - 2 + b, b)
    for b in range(2):
        wait_outs(b)


def kernel(input, table):
    idxT = input.astype(jnp.int32).T
    tableT = table.T
    tail1d = table[V - V_TAIL:, :].reshape(-1)
    packed, flatidx = _k1(tableT, idxT, tail1d)
    out1d = _k2(flatidx, packed.reshape(VPAD, D))
    return out1d.reshape(4, 6400, 8, 128).transpose(1, 3, 0, 2).reshape(B, D)


# speedup vs baseline: 1.0661x; 1.0007x over previous
import functools

import jax
import jax.numpy as jnp
from jax import lax
from jax.experimental import pallas as pl
from jax.experimental.pallas import tpu as pltpu
from jax.experimental.pallas import tpu_sc as plsc

NC, NS = 2, 16
NW = NC * NS
V = 1000001
D = 32
B = 16384 * 50
VT_FULL = V // 128
V_TAIL = V - VT_FULL * 128
VPAD = 1000004
SLAB = 256
SPW = 7808 // (SLAB // 128) // NW
REM1 = VT_FULL - 7808
RT = 16384 // 128
B_PER_W = B // NW
CH = 512
NCH = B_PER_W // CH
TPW2 = B_PER_W // 128
TILE = 128 * D

_mesh = plsc.VectorSubcoreMesh(core_axis_name="c", subcore_axis_name="s")


def _wid():
    return lax.axis_index("s") * NC + lax.axis_index("c")


def _iota():
    return lax.iota(jnp.int32, 16)


@functools.partial(
    pl.kernel,
    out_type=(
        jax.ShapeDtypeStruct((VPAD * D,), jnp.float32),
        jax.ShapeDtypeStruct((B,), jnp.int32),
    ),
    mesh=_mesh,
    compiler_params=pltpu.CompilerParams(
        use_tc_tiling_on_sc=True, needs_layout_passes=False
    ),
    scratch_types=[
        pltpu.VMEM((D, SLAB), jnp.float32),
        pltpu.VMEM((D, SLAB), jnp.float32),
        pltpu.VMEM((SLAB * D,), jnp.float32),
        pltpu.VMEM((SLAB * D,), jnp.float32),
        pltpu.VMEM((50, 128), jnp.int32),
        pltpu.VMEM((128 * 50,), jnp.int32),
        pltpu.SemaphoreType.DMA,
        pltpu.SemaphoreType.DMA,
    ],
)
def _k1(tableT, idxT, tail1d, packed, flatidx,
        tb0, tb1, pb0, pb1, ibuf, obuf, isem, osem):
    wid = _wid()
    i16 = _iota()
    perms = [(i16 + c) & 15 for c in range(16)]
    i32v = i16 * D
    widx = [i32v + perms[c] for c in range(16)]
    i50x = i16 * 50
    tbufs = (tb0, tb1)
    pbufs = (pb0, pb1)
    base = wid * SPW

    def fire_in(t, s):
        pltpu.async_copy(tableT.at[:, pl.ds(t * SLAB, SLAB)], tbufs[s], isem)

    def wait_in(s):
        pltpu.make_async_copy(
            tableT.at[:, pl.ds(0, SLAB)], tbufs[s], isem
        ).wait()

    def fire_out(t, s):
        pltpu.async_copy(
            pbufs[s], packed.at[pl.ds(t * (SLAB * D), SLAB * D)], osem
        )

    def wait_out(s):
        pltpu.make_async_copy(
            pbufs[s], packed.at[pl.ds(0, SLAB * D)], osem
        ).wait()

    def transpose_slab(s, nlanes):
        @pl.loop(0, nlanes, step=16, unroll=2)
        def _(v0):
            cols = i16 + v0
            for f0 in (0, 16):
                for c in range(16):
                    x = plsc.load_gather(tbufs[s], [perms[c] + f0, cols])
                    plsc.store_scatter(
                        pbufs[s], [widx[c] + (v0 * D + f0)], x
                    )

    for s in range(2):
        fire_in(base + s, s)
    for s in range(2):
        wait_in(s)
        transpose_slab(s, SLAB)
        fire_out(base + s, s)
        fire_in(base + s + 2, s)

    @pl.loop(2, SPW - 2, step=2)
    def _(j):
        for s in range(2):
            t = base + j + s
            wait_in(s)
            wait_out(s)
            transpose_slab(s, SLAB)
            fire_out(t, s)
            fire_in(t + 2, s)

    for s in range(2):
        wait_in(s)
        wait_out(s)
        transpose_slab(s, SLAB)
        fire_out(base + (SPW - 2) + s, s)
    for s in range(2):
        wait_out(s)

    @pl.when(wid < REM1)
    def _():
        t = NW * SPW * (SLAB // 128) + wid
        pltpu.sync_copy(
            tableT.at[:, pl.ds(t * 128, 128)], tb0.at[:, pl.ds(0, 128)]
        )
        transpose_slab(0, 128)
        pltpu.sync_copy(
            pb0.at[pl.ds(0, TILE)], packed.at[pl.ds(t * TILE, TILE)]
        )

    @pl.when(wid == REM1)
    def _():
        pltpu.sync_copy(tail1d, pb0.at[pl.ds(0, V_TAIL * D)])
        pltpu.sync_copy(
            pb0.at[pl.ds(0, (VPAD - VT_FULL * 128) * D)],
            packed.at[pl.ds(VT_FULL * 128 * D, (VPAD - VT_FULL * 128) * D)],
        )

    @pl.loop(0, RT // NW)
    def _(j):
        t = wid * (RT // NW) + j
        pltpu.sync_copy(idxT.at[:, pl.ds(t * 128, 128)], ibuf)

        @pl.loop(0, 8)
        def _(vv0):
            for c in range(50):
                x = ibuf[c, pl.ds(vv0 * 16, 16)]
                plsc.store_scatter(obuf, [i50x + (vv0 * (16 * 50) + c)], x)

        pltpu.sync_copy(obuf, flatidx.at[pl.ds(t * 6400, 6400)])


@functools.partial(
    pl.kernel,
    out_type=jax.ShapeDtypeStruct((B * D,), jnp.float32),
    mesh=_mesh,
    compiler_params=pltpu.CompilerParams(
        use_tc_tiling_on_sc=False, needs_layout_passes=False
    ),
    scratch_types=[
        pltpu.VMEM((CH,), jnp.int32),
        pltpu.VMEM((CH,), jnp.int32),
        pltpu.VMEM((CH, D), jnp.float32),
        pltpu.VMEM((CH, D), jnp.float32),
        pltpu.VMEM((CH * D,), jnp.float32),
        pltpu.VMEM((CH * D,), jnp.float32),
        pltpu.SemaphoreType.DMA,
        pltpu.SemaphoreType.DMA,
        pltpu.SemaphoreType.DMA,
    ],
)
def _k2(flatidx, table2d, out1d, ix0, ix1, gb0, gb1, tp0, tp1,
        gsem, osem, xsem):
    wid = _wid()
    base = wid * B_PER_W
    i16 = _iota()
    perms = [(i16 + c) & 15 for c in range(16)]
    GSTR = (CH // 128) * 1024
    dsts = [((perms[c] >> 3) * GSTR) + ((perms[c] & 7) << 7) for c in range(16)]
    ixs = (ix0, ix1)
    gbs = (gb0, gb1)
    tps = (tp0, tp1)

    def fire_idx(j, b):
        pltpu.async_copy(flatidx.at[pl.ds(base + j * CH, CH)], ixs[b], xsem)

    def fire_gather(b):
        pltpu.make_async_copy(
            flatidx.at[pl.ds(0, CH)], ixs[b], xsem
        ).wait()
        pltpu.async_copy(table2d.at[ixs[b]], gbs[b], gsem)

    def drain_gather(b):
        pltpu.make_async_copy(table2d.at[ixs[b]], gbs[b], gsem).wait()

    def transpose_chunk(b):
        @pl.loop(0, CH, step=16, unroll=2)
        def _(k0):
            rows = i16 + k0
            koff = i16 + (((k0 >> 7) * 1024) + (k0 & 127))
            koff8 = koff + 2 * GSTR
            for f0 in (0, 16):
                for c in range(16):
                    x = plsc.load_gather(gbs[b], [rows, perms[c] + f0])
                    plsc.store_scatter(
                        tps[b], [dsts[c] + (koff if f0 == 0 else koff8)], x
                    )

    def fire_outs(j, b):
        t0 = wid * TPW2 + j * (CH // 128)
        for g in range(4):
            pltpu.async_copy(
                tps[b].at[pl.ds(g * GSTR, GSTR)],
                out1d.at[pl.ds((g * 6400 + t0) * 1024, GSTR)],
                osem,
            )

    def wait_outs(b):
        for g in range(4):
            pltpu.make_async_copy(
                tps[b].at[pl.ds(0, GSTR)], out1d.at[pl.ds(0, GSTR)], osem
            ).wait()

    for b in range(2):
        fire_idx(b, b)
    for b in range(2):
        fire_gather(b)
    for b in range(2):
        drain_gather(b)
        fire_idx(b + 2, b)
        transpose_chunk(b)
        fire_outs(b, b)
        fire_gather(b)

    @pl.loop(2, NCH - 2, step=2)
    def _(j):
        for b in range(2):
            drain_gather(b)
            fire_idx(j + b + 2, b)
            wait_outs(b)
            transpose_chunk(b)
            fire_outs(j + b, b)
            fire_gather(b)

    for b in range(2):
        drain_gather(b)
        wait_outs(b)
        transpose_chunk(b)
        fire_outs(NCH - 2 + b, b)
    for b in range(2):
        wait_outs(b)


def kernel(input, table):
    idxT = input.astype(jnp.int32).T
    tableT = table.T
    tail1d = table[V - V_TAIL:, :].reshape(-1)
    packed, flatidx = _k1(tableT, idxT, tail1d)
    out1d = _k2(flatidx, packed.reshape(VPAD, D))
    return out1d.reshape(4, 6400, 8, 128).transpose(1, 3, 0, 2).reshape(B, D)
